# Initial kernel scaffold; baseline (speedup 1.0000x reference)
import jax
import jax.numpy as jnp
from jax.experimental import pallas as pl


def kernel(feat, label, edge_index, enc_mask_token, W1, b1, gamma1, beta1, a1, W2, b2, gamma2, beta2, a2, Wp1, bp1, Wp2, bp2):
    raise NotImplementedError("write your pallas kernel here")


# speedup vs baseline: 1.4357x; 1.4357x over previous
import functools

import jax
import jax.numpy as jnp
from jax import lax
from jax.experimental import pallas as pl

N = 10000
E = 320000
D_IN = 128
HID = 256
D_OUT = 128
RATE = 0.5
TAU = 0.5
EPS_BN = 1e-5
EPS_NORM = 1e-12
NM = int(RATE * N)


_BI = 512
_BJ = 512


def _sim_body(z1i_ref, z2i_ref, z1j_ref, z2j_ref, lab_ref, out_ref):
    j = pl.program_id(1)
    a1 = z1i_ref[...]
    a2 = z2i_ref[...]
    b1 = z1j_ref[...]
    b2 = z2j_ref[...]
    col = j * _BJ + lax.broadcasted_iota(jnp.int32, (1, _BJ), 1)
    valid = col < NM
    lab = jnp.where(valid, lab_ref[...], 0.0)

    inv_tau = 1.0 / TAU

    def emat(a, b):
        p = jnp.exp(
            lax.dot_general(a, b, (((1,), (1,)), ((), ())),
                            preferred_element_type=jnp.float32) * inv_tau)
        return jnp.where(valid, p, 0.0)

    p11 = emat(a1, b1)
    p12 = emat(a1, b2)
    p22 = emat(a2, b2)
    p21 = emat(a2, b1)
    rows = jnp.stack([
        jnp.sum(p11, axis=1), jnp.sum(p11 * lab, axis=1),
        jnp.sum(p12, axis=1), jnp.sum(p12 * lab, axis=1),
        jnp.sum(p22, axis=1), jnp.sum(p22 * lab, axis=1),
        jnp.sum(p21, axis=1), jnp.sum(p21 * lab, axis=1),
    ])

    @pl.when(j == 0)
    def _():
        out_ref[...] = jnp.zeros_like(out_ref)

    out_ref[...] += rows


def _sim_sums(z1, z2, lab):
    ni = pl.cdiv(NM, _BI)
    nj = pl.cdiv(NM, _BJ)
    return pl.pallas_call(
        _sim_body,
        grid=(ni, nj),
        in_specs=[
            pl.BlockSpec((_BI, D_OUT), lambda i, j: (i, 0)),
            pl.BlockSpec((_BI, D_OUT), lambda i, j: (i, 0)),
            pl.BlockSpec((_BJ, D_OUT), lambda i, j: (j, 0)),
            pl.BlockSpec((_BJ, D_OUT), lambda i, j: (j, 0)),
            pl.BlockSpec((_BI, _BJ), lambda i, j: (i, j)),
        ],
        out_specs=pl.BlockSpec((8, _BI), lambda i, j: (0, i)),
        out_shape=jax.ShapeDtypeStruct((8, NM), jnp.float32),
    )(z1, z2, z1, z2, lab)


def _bn(x, gamma, beta):
    mean = jnp.mean(x, axis=0)
    var = jnp.mean((x - mean) ** 2, axis=0)
    return gamma * (x - mean) / jnp.sqrt(var + EPS_BN) + beta


def _prelu(x, a):
    return jnp.where(x >= 0, x, a * x)


def _l2norm_rows(z):
    n = jnp.sqrt(jnp.sum(z * z, axis=1, keepdims=True))
    return z / jnp.maximum(n, EPS_NORM)


def kernel(feat, label, edge_index, enc_mask_token,
           W1, b1, gamma1, beta1, a1,
           W2, b2, gamma2, beta2, a2,
           Wp1, bp1, Wp2, bp2):
    src = edge_index[0]
    dst = edge_index[1]

    perm = jax.random.permutation(jax.random.key(1), N)
    mask = jnp.sort(perm[:NM]).astype(jnp.int32)
    is_masked = jnp.zeros((N,), jnp.float32).at[mask].set(1.0)

    x = jnp.where(is_masked[:, None] > 0, enc_mask_token[0][None, :], feat)

    ones = jnp.ones((E,), jnp.float32)
    out_deg = jax.ops.segment_sum(ones, src, num_segments=N)
    in_deg = jax.ops.segment_sum(ones, dst, num_segments=N)
    norm_src = jnp.clip(out_deg, 1.0, None) ** (-0.5)
    norm_dst = jnp.clip(in_deg, 1.0, None) ** (-0.5)

    def mp(h):
        msgs = jnp.take(h * norm_src[:, None], src, axis=0)
        return jax.ops.segment_sum(msgs, dst, num_segments=N)

    def enc(h0):
        g = mp(h0) * norm_dst[:, None] @ W1 + b1
        g = _prelu(_bn(g, gamma1, beta1), a1)
        u = g @ W2
        h = mp(u) * norm_dst[:, None] + b2
        return _prelu(_bn(h, gamma2, beta2), a2)

    h1 = enc(x)
    h2 = enc(feat)

    def proj(h):
        return jnp.maximum(h @ Wp1 + bp1, 0.0) @ Wp2 + bp2

    c_h = proj(jnp.take(h2, mask, axis=0))
    c_m = proj(jnp.take(h1, mask, axis=0))
    z1 = _l2norm_rows(c_h)
    z2 = _l2norm_rows(c_m)

    lab = jnp.take(jnp.take(label, mask, axis=0), mask, axis=1)

    sums = _sim_sums(z1, z2, lab)
    s11, l11, s12, l12, s22, l22, s21, l21 = [sums[k] for k in range(8)]
    d1 = jnp.exp(jnp.sum(z1 * z1, axis=1) / TAU)
    d2 = jnp.exp(jnp.sum(z2 * z2, axis=1) / TAU)

    loss1 = -jnp.log((l12 + l11 - d1) / (s11 + s12 - d1))
    loss2 = -jnp.log((l21 + l22 - d2) / (s22 + s21 - d2))
    return jnp.mean((loss1 + loss2) * 0.5)


# speedup vs baseline: 4.3197x; 3.0088x over previous
import functools

import jax
import jax.numpy as jnp
from jax import lax
from jax.experimental import pallas as pl
from jax.experimental.pallas import tpu as pltpu
from jax.experimental.pallas import tpu_sc as plsc

N = 10000
E = 320000
D_IN = 128
HID = 256
D_OUT = 128
RATE = 0.5
TAU = 0.5
EPS_BN = 1e-5
EPS_NORM = 1e-12
NM = int(RATE * N)

_NC = 2
_NS = 16
_NW = _NC * _NS
_EW = E // _NW
_KC = 80
_NCH = _EW // _KC
_GRP = 2
_DGRP = 5
_NGR = _NCH // _DGRP
_HN = N // 2
_AN = _HN + _NS
_ES = E // _NS
_MPCH = _ES // _KC
_NMPG = _MPCH // _GRP

_sc_mesh = plsc.VectorSubcoreMesh(core_axis_name="c", subcore_axis_name="s")


def _deg_body(sidx_hbm, didx_hbm, out_src, out_dst, sidx_v, didx_v,
              hist_src, hist_dst):
    c = lax.axis_index("c")
    s = lax.axis_index("s")
    w = c * _NS + s
    pltpu.sync_copy(sidx_hbm.at[w], sidx_v)
    pltpu.sync_copy(didx_hbm.at[w], didx_v)

    z16 = jnp.zeros((16,), jnp.float32)

    def zero_body(k, _):
        hist_src[pl.ds(k * 16, 16)] = z16
        hist_dst[pl.ds(k * 16, 16)] = z16
        return 0

    lax.fori_loop(0, N // 16, zero_body, 0)

    ones16 = jnp.ones((16,), jnp.float32)

    def body(i, _):
        for k in range(_KC // 16):
            sv = sidx_v[i, pl.ds(k * 16, 16)]
            plsc.addupdate_scatter(hist_src, [sv], ones16)
            dv = didx_v[i, pl.ds(k * 16, 16)]
            plsc.addupdate_scatter(hist_dst, [dv], ones16)
        return 0

    lax.fori_loop(0, _NCH, body, 0)
    pltpu.sync_copy(hist_src, out_src.at[w])
    pltpu.sync_copy(hist_dst, out_dst.at[w])


_deg_call = pl.kernel(
    _deg_body, mesh=_sc_mesh,
    compiler_params=pltpu.CompilerParams(needs_layout_passes=False),
    out_type=[jax.ShapeDtypeStruct((_NW, N), jnp.float32),
              jax.ShapeDtypeStruct((_NW, N), jnp.float32)],
    scratch_types=[
        pltpu.VMEM((_NCH, _KC), jnp.int32),
        pltpu.VMEM((_NCH, _KC), jnp.int32),
        pltpu.VMEM((N,), jnp.float32),
        pltpu.VMEM((N,), jnp.float32),
    ],
)


def _chunked(total):
    out = []
    o = 0
    while o < total:
        sz = min(_KC, total - o)
        out.append((o, sz))
        o += sz
    return out


def _mp_body(h_hbm, sidx_hbm, didx_hbm, zeros_hbm, out_hbm,
             sidx_v, didx_v, b0, b1, acc0, s0, s1):
    c = lax.axis_index("c")
    s = lax.axis_index("s")
    bufs = (b0, b1)
    sems = (s0, s1)
    acc = acc0
    pltpu.sync_copy(sidx_hbm.at[s], sidx_v)
    pltpu.sync_copy(didx_hbm.at[s], didx_v)

    lo = c * _HN
    garbage = _HN + s

    def xform(i, _):
        for k in range(_KC // 16):
            v = didx_v[i, pl.ds(k * 16, 16)] - lo
            ok = (v >= 0) & (v < _HN)
            didx_v[i, pl.ds(k * 16, 16)] = jnp.where(ok, v, garbage)
        return 0

    lax.fori_loop(0, _MPCH, xform, 0)

    zbase = s * 312

    def zero_acc():
        @pl.when(s < _NS - 1)
        def _():
            for o, sz in _chunked(312):
                pltpu.sync_copy(b0.at[pl.ds(0, sz)],
                                acc.at[pl.ds(zbase + o, sz)])

        @pl.when(s == _NS - 1)
        def _():
            for o, sz in _chunked(_AN - 15 * 312):
                pltpu.sync_copy(b0.at[pl.ds(0, sz)],
                                acc.at[pl.ds(zbase + o, sz)])

    def scatter_table(t):
        def body(g, _):
            descs = []
            for b in range(_GRP):
                descs.append(pltpu.async_copy(
                    h_hbm.at[t].at[sidx_v.at[g * _GRP + b]], bufs[b],
                    sems[b]))
            for b in range(_GRP):
                descs[b].wait()
            for b in range(_GRP):
                pltpu.sync_copy(bufs[b], acc.at[didx_v.at[g * _GRP + b]],
                                add=True)
            return 0

        lax.fori_loop(0, _NMPG, body, 0)

    def writeback(t):
        @pl.when(s < _NS - 1)
        def _():
            for o, sz in _chunked(312):
                pltpu.sync_copy(acc.at[pl.ds(zbase + o, sz)],
                                b1.at[pl.ds(0, sz)])
                pltpu.sync_copy(b1.at[pl.ds(0, sz)],
                                out_hbm.at[t, c, pl.ds(zbase + o, sz)])

        @pl.when(s == _NS - 1)
        def _():
            for o, sz in _chunked(_HN - 15 * 312):
                pltpu.sync_copy(acc.at[pl.ds(zbase + o, sz)],
                                b1.at[pl.ds(0, sz)])
                pltpu.sync_copy(b1.at[pl.ds(0, sz)],
                                out_hbm.at[t, c, pl.ds(zbase + o, sz)])

    pltpu.sync_copy(zeros_hbm, b0)
    zero_acc()
    plsc.subcore_barrier()
    scatter_table(0)
    plsc.subcore_barrier()
    writeback(0)
    zero_acc()
    plsc.subcore_barrier()
    scatter_table(1)
    plsc.subcore_barrier()
    writeback(1)


_mp_call = pl.kernel(
    _mp_body, mesh=_sc_mesh,
    out_type=jax.ShapeDtypeStruct((2, _NC, _HN, D_IN), jnp.float32),
    scratch_types=[
        pltpu.VMEM((_MPCH, _KC), jnp.int32),
        pltpu.VMEM((_MPCH, _KC), jnp.int32),
        pltpu.VMEM((_KC, D_IN), jnp.float32),
        pltpu.VMEM((_KC, D_IN), jnp.float32),
        pltpu.VMEM_SHARED((_AN, D_IN), jnp.float32),
        pltpu.SemaphoreType.DMA,
        pltpu.SemaphoreType.DMA,
    ],
)


_LP = 5120
_NGRP8 = NM // 8


_WID = (N // 128) * 128
_TB = N - 128
_SAFE = ((_TB - NM) // 16) * 16


def _lab_body(label_hbm, ltail_hbm, maskp_hbm, lab_hbm,
              maskp_v, rows8, tail8, outblk, sem, semt):
    c = lax.axis_index("c")
    s = lax.axis_index("s")
    w = c * _NS + s
    pltpu.sync_copy(maskp_hbm, maskp_v)
    lane = lax.broadcasted_iota(jnp.int32, (16,), 0)

    def group(k, _):
        g = w + k * _NW

        @pl.when(g < _NGRP8)
        def _():
            idx8 = maskp_v.at[pl.ds(g * 8, 8)]
            da = pltpu.async_copy(label_hbm.at[idx8, pl.ds(0, _WID)],
                                  rows8, sem)
            db = pltpu.async_copy(ltail_hbm.at[idx8], tail8, semt)
            da.wait()
            db.wait()
            for rr in range(8):
                rbase = jnp.full((16,), rr, jnp.int32)

                def colloop(j, _):
                    for u in range(4):
                        cidx = maskp_v[pl.ds((j * 4 + u) * 16, 16)]
                        vals = plsc.load_gather(rows8, [rbase, cidx])
                        outblk[rr, pl.ds((j * 4 + u) * 16, 16)] = vals
                    return 0

                lax.fori_loop(0, _SAFE // 64, colloop, 0)

                def colloop2(j, _):
                    cidx = maskp_v[pl.ds(j * 16, 16)]
                    wi = jnp.minimum(cidx, _WID - 1)
                    ti = jnp.maximum(cidx, _TB) - _TB
                    va = plsc.load_gather(rows8, [rbase, wi])
                    vb = plsc.load_gather(tail8, [rbase, ti])
                    vals = jnp.where(cidx < _WID, va, vb)
                    outblk[rr, pl.ds(j * 16, 16)] = vals
                    return 0

                lax.fori_loop(_SAFE // 16, NM // 16, colloop2, 0)
                cidx = maskp_v[pl.ds(NM - NM % 16, 16)]
                wi = jnp.minimum(cidx, _WID - 1)
                ti = jnp.maximum(cidx, _TB) - _TB
                va = plsc.load_gather(rows8, [rbase, wi])
                vb = plsc.load_gather(tail8, [rbase, ti])
                vals = jnp.where(cidx < _WID, va, vb)
                plsc.store_scatter(outblk, [rbase, (NM - NM % 16) + lane],
                                   vals, mask=lane < (NM % 16))
            pltpu.sync_copy(outblk, lab_hbm.at[pl.ds(g * 8, 8)])
        return 0

    lax.fori_loop(0, (_NGRP8 + _NW - 1) // _NW, group, 0)


_lab_call = pl.kernel(
    _lab_body, mesh=_sc_mesh,
    compiler_params=pltpu.CompilerParams(needs_layout_passes=False),
    out_type=jax.ShapeDtypeStruct((NM, NM), jnp.float32),
    scratch_types=[
        pltpu.VMEM((_LP,), jnp.int32),
        pltpu.VMEM((8, _WID), jnp.float32),
        pltpu.VMEM((8, 128), jnp.float32),
        pltpu.VMEM((8, NM), jnp.float32),
        pltpu.SemaphoreType.DMA,
        pltpu.SemaphoreType.DMA,
    ],
)


def _hrows_body(h1_hbm, h2_hbm, maskp_hbm, o1_hbm, o2_hbm,
                idx_v, buf, sem):
    c = lax.axis_index("c")
    s = lax.axis_index("s")
    w = c * _NS + s
    rows_per_w = _LP // _NW
    base = w * rows_per_w
    pltpu.sync_copy(maskp_hbm.at[pl.ds(base, rows_per_w)], idx_v)
    for t, (h, o) in enumerate(((h1_hbm, o1_hbm), (h2_hbm, o2_hbm))):
        for half in range(rows_per_w // _KC):
            pltpu.async_copy(
                h.at[idx_v.at[pl.ds(half * _KC, _KC)]], buf, sem).wait()
            pltpu.sync_copy(buf, o.at[pl.ds(base + half * _KC, _KC)])


_hrows_call = pl.kernel(
    _hrows_body, mesh=_sc_mesh,
    out_type=[jax.ShapeDtypeStruct((_LP, D_OUT), jnp.float32),
              jax.ShapeDtypeStruct((_LP, D_OUT), jnp.float32)],
    scratch_types=[
        pltpu.VMEM((_LP // _NW,), jnp.int32),
        pltpu.VMEM((_KC, D_OUT), jnp.float32),
        pltpu.SemaphoreType.DMA,
    ],
)


_BI = 512
_BJ = 512


def _sim_body(z1i_ref, z2i_ref, z1j_ref, z2j_ref, lab_ref, out_ref):
    j = pl.program_id(1)
    a1 = z1i_ref[...]
    a2 = z2i_ref[...]
    b1 = z1j_ref[...]
    b2 = z2j_ref[...]
    col = j * _BJ + lax.broadcasted_iota(jnp.int32, (1, _BJ), 1)
    valid = col < NM
    lab = jnp.where(valid, lab_ref[...], 0.0)

    inv_tau = 1.0 / TAU

    def emat(a, b):
        p = jnp.exp(
            lax.dot_general(a, b, (((1,), (1,)), ((), ())),
                            preferred_element_type=jnp.float32) * inv_tau)
        return jnp.where(valid, p, 0.0)

    p11 = emat(a1, b1)
    p12 = emat(a1, b2)
    p22 = emat(a2, b2)
    p21 = emat(a2, b1)
    rows = jnp.stack([
        jnp.sum(p11, axis=1), jnp.sum(p11 * lab, axis=1),
        jnp.sum(p12, axis=1), jnp.sum(p12 * lab, axis=1),
        jnp.sum(p22, axis=1), jnp.sum(p22 * lab, axis=1),
        jnp.sum(p21, axis=1), jnp.sum(p21 * lab, axis=1),
    ])

    @pl.when(j == 0)
    def _():
        out_ref[...] = jnp.zeros_like(out_ref)

    out_ref[...] += rows


def _sim_sums(z1, z2, lab):
    ni = pl.cdiv(NM, _BI)
    nj = pl.cdiv(NM, _BJ)
    return pl.pallas_call(
        _sim_body,
        grid=(ni, nj),
        in_specs=[
            pl.BlockSpec((_BI, D_OUT), lambda i, j: (i, 0)),
            pl.BlockSpec((_BI, D_OUT), lambda i, j: (i, 0)),
            pl.BlockSpec((_BJ, D_OUT), lambda i, j: (j, 0)),
            pl.BlockSpec((_BJ, D_OUT), lambda i, j: (j, 0)),
            pl.BlockSpec((_BI, _BJ), lambda i, j: (i, j)),
        ],
        out_specs=pl.BlockSpec((8, _BI), lambda i, j: (0, i)),
        out_shape=jax.ShapeDtypeStruct((8, NM), jnp.float32),
    )(z1, z2, z1, z2, lab)


def _bn(x, gamma, beta):
    mean = jnp.mean(x, axis=0)
    var = jnp.mean((x - mean) ** 2, axis=0)
    return gamma * (x - mean) / jnp.sqrt(var + EPS_BN) + beta


def _prelu(x, a):
    return jnp.where(x >= 0, x, a * x)


def _l2norm_rows(z):
    n = jnp.sqrt(jnp.sum(z * z, axis=1, keepdims=True))
    return z / jnp.maximum(n, EPS_NORM)


def kernel(feat, label, edge_index, enc_mask_token,
           W1, b1, gamma1, beta1, a1,
           W2, b2, gamma2, beta2, a2,
           Wp1, bp1, Wp2, bp2):
    src = edge_index[0]
    dst = edge_index[1]

    perm = jax.random.permutation(jax.random.key(1), N)
    mask = jnp.sort(perm[:NM]).astype(jnp.int32)
    is_masked = jnp.zeros((N,), jnp.float32).at[mask].set(1.0)

    x = jnp.where(is_masked[:, None] > 0, enc_mask_token[0][None, :], feat)

    sidx32 = src.reshape(_NW, _NCH, _KC)
    didx32 = dst.reshape(_NW, _NCH, _KC)
    hs, hd = _deg_call(sidx32, didx32)
    out_deg = jnp.sum(hs, axis=0)
    in_deg = jnp.sum(hd, axis=0)
    norm_src = jnp.clip(out_deg, 1.0, None) ** (-0.5)
    norm_dst = jnp.clip(in_deg, 1.0, None) ** (-0.5)

    sidx16 = src.reshape(_NS, _MPCH, _KC)
    didx16 = dst.reshape(_NS, _MPCH, _KC)
    mp_zeros = jnp.zeros((_KC, D_IN), jnp.float32)

    def bn_b(z, gamma, beta):
        mean = jnp.mean(z, axis=1, keepdims=True)
        var = jnp.mean((z - mean) ** 2, axis=1, keepdims=True)
        return gamma * (z - mean) / jnp.sqrt(var + EPS_BN) + beta

    ns = norm_src[None, :, None]
    nd = norm_dst[None, :, None]
    tables0 = jnp.stack([x, feat]) * ns

    def scan_body(tables, l):
        parts = _mp_call(tables, sidx16, didx16, mp_zeros)
        agg = jnp.concatenate([parts[:, 0], parts[:, 1]], axis=1) * nd

        def layer1(_):
            g = jnp.einsum('bnd,dh->bnh', agg, W1,
                           preferred_element_type=jnp.float32) + b1
            g = _prelu(bn_b(g, gamma1, beta1), a1)
            u = jnp.einsum('bnh,hd->bnd', g, W2,
                           preferred_element_type=jnp.float32) * ns
            return u, jnp.zeros((2, N, D_OUT), jnp.float32)

        def layer2(_):
            h = agg + b2
            return tables, _prelu(bn_b(h, gamma2, beta2), a2)

        return lax.cond(l == 0, layer1, layer2, None)

    _, outs = lax.scan(scan_body, tables0, jnp.arange(2))
    h1 = outs[1, 0]
    h2 = outs[1, 1]

    def proj(h):
        return jnp.maximum(h @ Wp1 + bp1, 0.0) @ Wp2 + bp2

    maskp = jnp.concatenate(
        [mask, jnp.zeros((_LP - NM,), jnp.int32)])
    g1, g2 = _hrows_call(h1, h2, maskp)
    c_h = proj(g2[:NM])
    c_m = proj(g1[:NM])
    z1 = _l2norm_rows(c_h)
    z2 = _l2norm_rows(c_m)

    ltail = lax.slice(label, (0, _TB), (N, N))
    lab = _lab_call(label, ltail, maskp)

    sums = _sim_sums(z1, z2, lab)
    s11, l11, s12, l12, s22, l22, s21, l21 = [sums[k] for k in range(8)]
    d1 = jnp.exp(jnp.sum(z1 * z1, axis=1) / TAU)
    d2 = jnp.exp(jnp.sum(z2 * z2, axis=1) / TAU)

    loss1 = -jnp.log((l12 + l11 - d1) / (s11 + s12 - d1))
    loss2 = -jnp.log((l21 + l22 - d2) / (s22 + s21 - d2))
    return jnp.mean((loss1 + loss2) * 0.5)


# speedup vs baseline: 4.9472x; 1.1453x over previous
import functools

import jax
import jax.numpy as jnp
from jax import lax
from jax.experimental import pallas as pl
from jax.experimental.pallas import tpu as pltpu
from jax.experimental.pallas import tpu_sc as plsc

N = 10000
E = 320000
D_IN = 128
HID = 256
D_OUT = 128
RATE = 0.5
TAU = 0.5
EPS_BN = 1e-5
EPS_NORM = 1e-12
NM = int(RATE * N)

_NC = 2
_NS = 16
_NW = _NC * _NS
_EW = E // _NW
_KC = 80
_NCH = _EW // _KC
_HN = N // 2
_AN = _HN + _NS
_ES = E // _NS
_MKC = 80
_MPCH = _ES // _MKC
_NPC = 10
_NPIECE = _MPCH // _NPC
_CAT = 2 * D_IN

_sc_mesh = plsc.VectorSubcoreMesh(core_axis_name="c", subcore_axis_name="s")


def _deg_body(sidx_hbm, didx_hbm, out_src, out_dst, sidx_v, didx_v,
              hist_src, hist_dst):
    c = lax.axis_index("c")
    s = lax.axis_index("s")
    w = c * _NS + s
    pltpu.sync_copy(sidx_hbm.at[w], sidx_v)
    pltpu.sync_copy(didx_hbm.at[w], didx_v)

    z16 = jnp.zeros((16,), jnp.float32)

    def zero_body(k, _):
        hist_src[pl.ds(k * 16, 16)] = z16
        hist_dst[pl.ds(k * 16, 16)] = z16
        return 0

    lax.fori_loop(0, N // 16, zero_body, 0)

    ones16 = jnp.ones((16,), jnp.float32)

    def body(i, _):
        for k in range(_KC // 16):
            sv = sidx_v[i, pl.ds(k * 16, 16)]
            plsc.addupdate_scatter(hist_src, [sv], ones16)
            dv = didx_v[i, pl.ds(k * 16, 16)]
            plsc.addupdate_scatter(hist_dst, [dv], ones16)
        return 0

    lax.fori_loop(0, _NCH, body, 0)
    pltpu.sync_copy(hist_src, out_src.at[w])
    pltpu.sync_copy(hist_dst, out_dst.at[w])


_deg_call = pl.kernel(
    _deg_body, mesh=_sc_mesh,
    compiler_params=pltpu.CompilerParams(needs_layout_passes=False),
    out_type=[jax.ShapeDtypeStruct((_NW, N), jnp.float32),
              jax.ShapeDtypeStruct((_NW, N), jnp.float32)],
    scratch_types=[
        pltpu.VMEM((_NCH, _KC), jnp.int32),
        pltpu.VMEM((_NCH, _KC), jnp.int32),
        pltpu.VMEM((N,), jnp.float32),
        pltpu.VMEM((N,), jnp.float32),
    ],
)


def _chunked(total, step=_KC):
    out = []
    o = 0
    while o < total:
        sz = min(step, total - o)
        out.append((o, sz))
        o += sz
    return out


def _chunked16(total):
    return _chunked(total, 16)


def _mp_body(h_hbm, sidx_hbm, didx_hbm, zeros_hbm, out_hbm,
             sv, dv, a0, a1, b0, b1, stage, acc0, acc1,
             sa0, sa1, sb0, sb1):
    c = lax.axis_index("c")
    s = lax.axis_index("s")
    accs = (acc0, acc1)
    lo = c * _HN
    garbage = _HN + s
    zbase = s * 312

    pltpu.sync_copy(zeros_hbm, stage)
    for acc in accs:

        @pl.when(s < _NS - 1)
        def _():
            for o, sz in _chunked16(312):
                pltpu.sync_copy(stage.at[pl.ds(0, sz)],
                                acc.at[pl.ds(zbase + o, sz)])

        @pl.when(s == _NS - 1)
        def _():
            for o, sz in _chunked16(_AN - 15 * 312):
                pltpu.sync_copy(stage.at[pl.ds(0, sz)],
                                acc.at[pl.ds(zbase + o, sz)])

    plsc.subcore_barrier()

    def gstart(piece_chunk, bufp, semp):
        for t in range(2):
            pltpu.async_copy(h_hbm.at[t].at[sv.at[piece_chunk]],
                             bufp[t], semp[t])

    def gwait(piece_chunk, bufp, semp):
        for t in range(2):
            pltpu.make_async_copy(h_hbm.at[t].at[sv.at[piece_chunk]],
                                  bufp[t], semp[t]).wait()

    def scat(piece_chunk, bufp):
        for t in range(2):
            pltpu.sync_copy(bufp[t], accs[t].at[dv.at[piece_chunk]],
                            add=True)

    A = (a0, a1)
    B = (b0, b1)
    SA = (sa0, sa1)
    SB = (sb0, sb1)

    def piece(p, _):
        pltpu.sync_copy(sidx_hbm.at[s, p], sv)
        pltpu.sync_copy(didx_hbm.at[s, p], dv)

        def xform(i, _):
            for k in range(_MKC // 16):
                v = dv[i, pl.ds(k * 16, 16)] - lo
                ok = (v >= 0) & (v < _HN)
                dv[i, pl.ds(k * 16, 16)] = jnp.where(ok, v, garbage)
            return 0

        lax.fori_loop(0, _NPC, xform, 0)

        gstart(0, A, SA)
        gstart(1, B, SB)

        def pair(j, _):
            i0 = j * 2
            gwait(i0, A, SA)
            scat(i0, A)

            @pl.when(i0 + 2 < _NPC)
            def _():
                gstart(i0 + 2, A, SA)

            gwait(i0 + 1, B, SB)
            scat(i0 + 1, B)

            @pl.when(i0 + 3 < _NPC)
            def _():
                gstart(i0 + 3, B, SB)

            return 0

        lax.fori_loop(0, _NPC // 2, pair, 0)
        if _NPC % 2:
            gwait(_NPC - 1, A, SA)
            scat(_NPC - 1, A)
        return 0

    lax.fori_loop(0, _NPIECE, piece, 0)
    plsc.subcore_barrier()

    def wb(o, sz):
        for t in range(2):
            pltpu.sync_copy(accs[t].at[pl.ds(zbase + o, sz)],
                            stage.at[pl.ds(0, sz)])
            pltpu.sync_copy(stage.at[pl.ds(0, sz)],
                            out_hbm.at[t, c, pl.ds(zbase + o, sz)])

    @pl.when(s < _NS - 1)
    def _():
        for o, sz in _chunked16(312):
            wb(o, sz)

    @pl.when(s == _NS - 1)
    def _():
        for o, sz in _chunked16(_HN - 15 * 312):
            wb(o, sz)


_mp_call = pl.kernel(
    _mp_body, mesh=_sc_mesh,
    out_type=jax.ShapeDtypeStruct((2, _NC, _HN, D_IN), jnp.float32),
    scratch_types=[
        pltpu.VMEM((_NPC, _MKC), jnp.int32),
        pltpu.VMEM((_NPC, _MKC), jnp.int32),
        pltpu.VMEM((_MKC, D_IN), jnp.float32),
        pltpu.VMEM((_MKC, D_IN), jnp.float32),
        pltpu.VMEM((_MKC, D_IN), jnp.float32),
        pltpu.VMEM((_MKC, D_IN), jnp.float32),
        pltpu.VMEM((16, D_IN), jnp.float32),
        pltpu.VMEM_SHARED((_AN, D_IN), jnp.float32),
        pltpu.VMEM_SHARED((_AN, D_IN), jnp.float32),
        pltpu.SemaphoreType.DMA,
        pltpu.SemaphoreType.DMA,
        pltpu.SemaphoreType.DMA,
        pltpu.SemaphoreType.DMA,
    ],
)


_LP = 5120
_NGRP8 = NM // 8


_WID = (N // 128) * 128
_TB = N - 128
_SAFE = ((_TB - NM) // 16) * 16


def _lab_body(label_hbm, ltail_hbm, maskp_hbm, lab_hbm,
              maskp_v, rows8, tail8, outblk, sem, semt):
    c = lax.axis_index("c")
    s = lax.axis_index("s")
    w = c * _NS + s
    pltpu.sync_copy(maskp_hbm, maskp_v)
    lane = lax.broadcasted_iota(jnp.int32, (16,), 0)

    def group(k, _):
        g = w + k * _NW

        @pl.when(g < _NGRP8)
        def _():
            idx8 = maskp_v.at[pl.ds(g * 8, 8)]
            da = pltpu.async_copy(label_hbm.at[idx8, pl.ds(0, _WID)],
                                  rows8, sem)
            db = pltpu.async_copy(ltail_hbm.at[idx8], tail8, semt)
            da.wait()
            db.wait()
            for rr in range(8):
                rbase = jnp.full((16,), rr, jnp.int32)

                def colloop(j, _):
                    for u in range(8):
                        cidx = maskp_v[pl.ds((j * 8 + u) * 16, 16)]
                        vals = plsc.load_gather(rows8, [rbase, cidx])
                        outblk[rr, pl.ds((j * 8 + u) * 16, 16)] = vals
                    return 0

                lax.fori_loop(0, _SAFE // 128, colloop, 0)

                def colloop2(j, _):
                    cidx = maskp_v[pl.ds(j * 16, 16)]
                    wi = jnp.minimum(cidx, _WID - 1)
                    ti = jnp.maximum(cidx, _TB) - _TB
                    va = plsc.load_gather(rows8, [rbase, wi])
                    vb = plsc.load_gather(tail8, [rbase, ti])
                    vals = jnp.where(cidx < _WID, va, vb)
                    outblk[rr, pl.ds(j * 16, 16)] = vals
                    return 0

                lax.fori_loop(_SAFE // 16, NM // 16, colloop2, 0)
                cidx = maskp_v[pl.ds(NM - NM % 16, 16)]
                wi = jnp.minimum(cidx, _WID - 1)
                ti = jnp.maximum(cidx, _TB) - _TB
                va = plsc.load_gather(rows8, [rbase, wi])
                vb = plsc.load_gather(tail8, [rbase, ti])
                vals = jnp.where(cidx < _WID, va, vb)
                plsc.store_scatter(outblk, [rbase, (NM - NM % 16) + lane],
                                   vals, mask=lane < (NM % 16))
            pltpu.sync_copy(outblk, lab_hbm.at[pl.ds(g * 8, 8)])
        return 0

    lax.fori_loop(0, (_NGRP8 + _NW - 1) // _NW, group, 0)


_lab_call = pl.kernel(
    _lab_body, mesh=_sc_mesh,
    compiler_params=pltpu.CompilerParams(needs_layout_passes=False),
    out_type=jax.ShapeDtypeStruct((NM, NM), jnp.float32),
    scratch_types=[
        pltpu.VMEM((_LP,), jnp.int32),
        pltpu.VMEM((8, _WID), jnp.float32),
        pltpu.VMEM((8, 128), jnp.float32),
        pltpu.VMEM((8, NM), jnp.float32),
        pltpu.SemaphoreType.DMA,
        pltpu.SemaphoreType.DMA,
    ],
)


def _hrows_body(h1_hbm, h2_hbm, maskp_hbm, o1_hbm, o2_hbm,
                idx_v, buf, sem):
    c = lax.axis_index("c")
    s = lax.axis_index("s")
    w = c * _NS + s
    rows_per_w = _LP // _NW
    base = w * rows_per_w
    pltpu.sync_copy(maskp_hbm.at[pl.ds(base, rows_per_w)], idx_v)
    for t, (h, o) in enumerate(((h1_hbm, o1_hbm), (h2_hbm, o2_hbm))):
        for half in range(rows_per_w // _KC):
            pltpu.async_copy(
                h.at[idx_v.at[pl.ds(half * _KC, _KC)]], buf, sem).wait()
            pltpu.sync_copy(buf, o.at[pl.ds(base + half * _KC, _KC)])


_hrows_call = pl.kernel(
    _hrows_body, mesh=_sc_mesh,
    out_type=[jax.ShapeDtypeStruct((_LP, D_OUT), jnp.float32),
              jax.ShapeDtypeStruct((_LP, D_OUT), jnp.float32)],
    scratch_types=[
        pltpu.VMEM((_LP // _NW,), jnp.int32),
        pltpu.VMEM((_KC, D_OUT), jnp.float32),
        pltpu.SemaphoreType.DMA,
    ],
)


_BI = 512
_BJ = 512


def _sim_body(z1i_ref, z2i_ref, z1j_ref, z2j_ref, lab_ref, out_ref):
    j = pl.program_id(1)
    a1 = z1i_ref[...]
    a2 = z2i_ref[...]
    b1 = z1j_ref[...]
    b2 = z2j_ref[...]
    col = j * _BJ + lax.broadcasted_iota(jnp.int32, (1, _BJ), 1)
    valid = col < NM
    lab = jnp.where(valid, lab_ref[...], 0.0)

    inv_tau = 1.0 / TAU

    def emat(a, b):
        p = jnp.exp(
            lax.dot_general(a, b, (((1,), (1,)), ((), ())),
                            preferred_element_type=jnp.float32) * inv_tau)
        return jnp.where(valid, p, 0.0)

    p11 = emat(a1, b1)
    p12 = emat(a1, b2)
    p22 = emat(a2, b2)
    p21 = emat(a2, b1)
    rows = jnp.stack([
        jnp.sum(p11, axis=1), jnp.sum(p11 * lab, axis=1),
        jnp.sum(p12, axis=1), jnp.sum(p12 * lab, axis=1),
        jnp.sum(p22, axis=1), jnp.sum(p22 * lab, axis=1),
        jnp.sum(p21, axis=1), jnp.sum(p21 * lab, axis=1),
    ])

    @pl.when(j == 0)
    def _():
        out_ref[...] = jnp.zeros_like(out_ref)

    out_ref[...] += rows


def _sim_sums(z1, z2, lab):
    ni = pl.cdiv(NM, _BI)
    nj = pl.cdiv(NM, _BJ)
    return pl.pallas_call(
        _sim_body,
        grid=(ni, nj),
        in_specs=[
            pl.BlockSpec((_BI, D_OUT), lambda i, j: (i, 0)),
            pl.BlockSpec((_BI, D_OUT), lambda i, j: (i, 0)),
            pl.BlockSpec((_BJ, D_OUT), lambda i, j: (j, 0)),
            pl.BlockSpec((_BJ, D_OUT), lambda i, j: (j, 0)),
            pl.BlockSpec((_BI, _BJ), lambda i, j: (i, j)),
        ],
        out_specs=pl.BlockSpec((8, _BI), lambda i, j: (0, i)),
        out_shape=jax.ShapeDtypeStruct((8, NM), jnp.float32),
    )(z1, z2, z1, z2, lab)


def _bn(x, gamma, beta):
    mean = jnp.mean(x, axis=0)
    var = jnp.mean((x - mean) ** 2, axis=0)
    return gamma * (x - mean) / jnp.sqrt(var + EPS_BN) + beta


def _prelu(x, a):
    return jnp.where(x >= 0, x, a * x)


def _l2norm_rows(z):
    n = jnp.sqrt(jnp.sum(z * z, axis=1, keepdims=True))
    return z / jnp.maximum(n, EPS_NORM)


def kernel(feat, label, edge_index, enc_mask_token,
           W1, b1, gamma1, beta1, a1,
           W2, b2, gamma2, beta2, a2,
           Wp1, bp1, Wp2, bp2):
    src = edge_index[0]
    dst = edge_index[1]

    perm = jax.random.permutation(jax.random.key(1), N)
    mask = jnp.sort(perm[:NM]).astype(jnp.int32)
    is_masked = jnp.zeros((N,), jnp.float32).at[mask].set(1.0)

    x = jnp.where(is_masked[:, None] > 0, enc_mask_token[0][None, :], feat)

    sidx32 = src.reshape(_NW, _NCH, _KC)
    didx32 = dst.reshape(_NW, _NCH, _KC)
    hs, hd = _deg_call(sidx32, didx32)
    out_deg = jnp.sum(hs, axis=0)
    in_deg = jnp.sum(hd, axis=0)
    norm_src = jnp.clip(out_deg, 1.0, None) ** (-0.5)
    norm_dst = jnp.clip(in_deg, 1.0, None) ** (-0.5)

    sidx16 = src.reshape(_NS, _NPIECE, _NPC, _MKC)
    didx16 = dst.reshape(_NS, _NPIECE, _NPC, _MKC)
    mp_zeros = jnp.zeros((16, D_IN), jnp.float32)

    def bn_b(z, gamma, beta):
        mean = jnp.mean(z, axis=1, keepdims=True)
        var = jnp.mean((z - mean) ** 2, axis=1, keepdims=True)
        return gamma * (z - mean) / jnp.sqrt(var + EPS_BN) + beta

    ns = norm_src[None, :, None]
    nd = norm_dst[None, :, None]
    tables0 = jnp.stack([x, feat]) * ns

    def scan_body(tables, l):
        parts = _mp_call(tables, sidx16, didx16, mp_zeros)
        agg = jnp.concatenate([parts[:, 0], parts[:, 1]], axis=1) * nd

        def layer1(_):
            g = jnp.einsum('bnd,dh->bnh', agg, W1,
                           preferred_element_type=jnp.float32) + b1
            g = _prelu(bn_b(g, gamma1, beta1), a1)
            u = jnp.einsum('bnh,hd->bnd', g, W2,
                           preferred_element_type=jnp.float32) * ns
            return u, jnp.zeros((2, N, D_OUT), jnp.float32)

        def layer2(_):
            h = agg + b2
            return tables, _prelu(bn_b(h, gamma2, beta2), a2)

        return lax.cond(l == 0, layer1, layer2, None)

    _, outs = lax.scan(scan_body, tables0, jnp.arange(2))
    h1 = outs[1, 0]
    h2 = outs[1, 1]

    def proj(h):
        return jnp.maximum(h @ Wp1 + bp1, 0.0) @ Wp2 + bp2

    maskp = jnp.concatenate(
        [mask, jnp.zeros((_LP - NM,), jnp.int32)])
    g1, g2 = _hrows_call(h1, h2, maskp)
    c_h = proj(g2[:NM])
    c_m = proj(g1[:NM])
    z1 = _l2norm_rows(c_h)
    z2 = _l2norm_rows(c_m)

    ltail = lax.slice(label, (0, _TB), (N, N))
    lab = _lab_call(label, ltail, maskp)

    sums = _sim_sums(z1, z2, lab)
    s11, l11, s12, l12, s22, l22, s21, l21 = [sums[k] for k in range(8)]
    d1 = jnp.exp(jnp.sum(z1 * z1, axis=1) / TAU)
    d2 = jnp.exp(jnp.sum(z2 * z2, axis=1) / TAU)

    loss1 = -jnp.log((l12 + l11 - d1) / (s11 + s12 - d1))
    loss2 = -jnp.log((l21 + l22 - d2) / (s22 + s21 - d2))
    return jnp.mean((loss1 + loss2) * 0.5)


# speedup vs baseline: 5.0938x; 1.0296x over previous
import functools

import jax
import jax.numpy as jnp
from jax import lax
from jax.experimental import pallas as pl
from jax.experimental.pallas import tpu as pltpu
from jax.experimental.pallas import tpu_sc as plsc

N = 10000
E = 320000
D_IN = 128
HID = 256
D_OUT = 128
RATE = 0.5
TAU = 0.5
EPS_BN = 1e-5
EPS_NORM = 1e-12
NM = int(RATE * N)

_NC = 2
_NS = 16
_NW = _NC * _NS
_EW = E // _NW
_KC = 80
_NCH = _EW // _KC
_HN = N // 2
_AN = _HN + _NS
_ES = E // _NS
_MKC = 80
_MPCH = _ES // _MKC
_NPC = 25
_NPIECE = _MPCH // _NPC
_CAT = 2 * D_IN

_sc_mesh = plsc.VectorSubcoreMesh(core_axis_name="c", subcore_axis_name="s")


def _deg_body(sidx_hbm, didx_hbm, out_src, out_dst, sidx_v, didx_v,
              hist_src, hist_dst):
    c = lax.axis_index("c")
    s = lax.axis_index("s")
    w = c * _NS + s
    pltpu.sync_copy(sidx_hbm.at[w], sidx_v)
    pltpu.sync_copy(didx_hbm.at[w], didx_v)

    z16 = jnp.zeros((16,), jnp.float32)

    def zero_body(k, _):
        hist_src[pl.ds(k * 16, 16)] = z16
        hist_dst[pl.ds(k * 16, 16)] = z16
        return 0

    lax.fori_loop(0, N // 16, zero_body, 0)

    ones16 = jnp.ones((16,), jnp.float32)

    def body(i, _):
        for k in range(_KC // 16):
            sv = sidx_v[i, pl.ds(k * 16, 16)]
            plsc.addupdate_scatter(hist_src, [sv], ones16)
            dv = didx_v[i, pl.ds(k * 16, 16)]
            plsc.addupdate_scatter(hist_dst, [dv], ones16)
        return 0

    lax.fori_loop(0, _NCH, body, 0)
    pltpu.sync_copy(hist_src, out_src.at[w])
    pltpu.sync_copy(hist_dst, out_dst.at[w])


_deg_call = pl.kernel(
    _deg_body, mesh=_sc_mesh,
    compiler_params=pltpu.CompilerParams(needs_layout_passes=False),
    out_type=[jax.ShapeDtypeStruct((_NW, N), jnp.float32),
              jax.ShapeDtypeStruct((_NW, N), jnp.float32)],
    scratch_types=[
        pltpu.VMEM((_NCH, _KC), jnp.int32),
        pltpu.VMEM((_NCH, _KC), jnp.int32),
        pltpu.VMEM((N,), jnp.float32),
        pltpu.VMEM((N,), jnp.float32),
    ],
)


def _chunked(total, step=_KC):
    out = []
    o = 0
    while o < total:
        sz = min(step, total - o)
        out.append((o, sz))
        o += sz
    return out


def _chunked8(total):
    return _chunked(total, 8)


def _mp_body(h_hbm, sidx_hbm, didx_hbm, zeros_hbm, out_hbm,
             sv, dv, a0, a1, b0, b1, stage, acc0, acc1,
             sa0, sa1, sb0, sb1):
    c = lax.axis_index("c")
    s = lax.axis_index("s")
    accs = (acc0, acc1)
    lo = c * _HN
    garbage = _HN + s
    zbase = s * 312

    pltpu.sync_copy(zeros_hbm, stage)
    for acc in accs:

        @pl.when(s < _NS - 1)
        def _():
            for o, sz in _chunked8(312):
                pltpu.sync_copy(stage.at[pl.ds(0, sz)],
                                acc.at[pl.ds(zbase + o, sz)])

        @pl.when(s == _NS - 1)
        def _():
            for o, sz in _chunked8(_AN - 15 * 312):
                pltpu.sync_copy(stage.at[pl.ds(0, sz)],
                                acc.at[pl.ds(zbase + o, sz)])

    plsc.subcore_barrier()

    def gstart(piece_chunk, bufp, semp):
        for t in range(2):
            pltpu.async_copy(h_hbm.at[t].at[sv.at[piece_chunk]],
                             bufp[t], semp[t])

    def gwait(piece_chunk, bufp, semp):
        for t in range(2):
            pltpu.make_async_copy(h_hbm.at[t].at[sv.at[piece_chunk]],
                                  bufp[t], semp[t]).wait()

    def scat(piece_chunk, bufp):
        for t in range(2):
            pltpu.sync_copy(bufp[t], accs[t].at[dv.at[piece_chunk]],
                            add=True)

    A = (a0, a1)
    B = (b0, b1)
    SA = (sa0, sa1)
    SB = (sb0, sb1)

    def piece(p, _):
        pltpu.sync_copy(sidx_hbm.at[s, p], sv)
        pltpu.sync_copy(didx_hbm.at[s, p], dv)

        def xform(i, _):
            for k in range(_MKC // 16):
                v = dv[i, pl.ds(k * 16, 16)] - lo
                ok = (v >= 0) & (v < _HN)
                dv[i, pl.ds(k * 16, 16)] = jnp.where(ok, v, garbage)
            return 0

        lax.fori_loop(0, _NPC, xform, 0)

        gstart(0, A, SA)
        gstart(1, B, SB)

        def pair(j, _):
            i0 = j * 2
            gwait(i0, A, SA)
            scat(i0, A)

            @pl.when(i0 + 2 < _NPC)
            def _():
                gstart(i0 + 2, A, SA)

            gwait(i0 + 1, B, SB)
            scat(i0 + 1, B)

            @pl.when(i0 + 3 < _NPC)
            def _():
                gstart(i0 + 3, B, SB)

            return 0

        lax.fori_loop(0, _NPC // 2, pair, 0)
        if _NPC % 2:
            gwait(_NPC - 1, A, SA)
            scat(_NPC - 1, A)
        return 0

    lax.fori_loop(0, _NPIECE, piece, 0)
    plsc.subcore_barrier()

    def wb(o, sz):
        for t in range(2):
            pltpu.sync_copy(accs[t].at[pl.ds(zbase + o, sz)],
                            stage.at[pl.ds(0, sz)])
            pltpu.sync_copy(stage.at[pl.ds(0, sz)],
                            out_hbm.at[t, c, pl.ds(zbase + o, sz)])

    @pl.when(s < _NS - 1)
    def _():
        for o, sz in _chunked8(312):
            wb(o, sz)

    @pl.when(s == _NS - 1)
    def _():
        for o, sz in _chunked8(_HN - 15 * 312):
            wb(o, sz)


_mp_call = pl.kernel(
    _mp_body, mesh=_sc_mesh,
    out_type=jax.ShapeDtypeStruct((2, _NC, _HN, D_IN), jnp.float32),
    scratch_types=[
        pltpu.VMEM((_NPC, _MKC), jnp.int32),
        pltpu.VMEM((_NPC, _MKC), jnp.int32),
        pltpu.VMEM((_MKC, D_IN), jnp.float32),
        pltpu.VMEM((_MKC, D_IN), jnp.float32),
        pltpu.VMEM((_MKC, D_IN), jnp.float32),
        pltpu.VMEM((_MKC, D_IN), jnp.float32),
        pltpu.VMEM((8, D_IN), jnp.float32),
        pltpu.VMEM_SHARED((_AN, D_IN), jnp.float32),
        pltpu.VMEM_SHARED((_AN, D_IN), jnp.float32),
        pltpu.SemaphoreType.DMA,
        pltpu.SemaphoreType.DMA,
        pltpu.SemaphoreType.DMA,
        pltpu.SemaphoreType.DMA,
    ],
)


_LP = 5120
_NGRP8 = NM // 8


_WID = (N // 128) * 128
_TB = N - 128
_SAFE = ((_TB - NM) // 16) * 16


def _lab_body(label_hbm, ltail_hbm, maskp_hbm, lab_hbm,
              maskp_v, rows8, tail8, outblk, sem, semt):
    c = lax.axis_index("c")
    s = lax.axis_index("s")
    w = c * _NS + s
    pltpu.sync_copy(maskp_hbm, maskp_v)
    lane = lax.broadcasted_iota(jnp.int32, (16,), 0)

    def group(k, _):
        g = w + k * _NW

        @pl.when(g < _NGRP8)
        def _():
            idx8 = maskp_v.at[pl.ds(g * 8, 8)]
            da = pltpu.async_copy(label_hbm.at[idx8, pl.ds(0, _WID)],
                                  rows8, sem)
            db = pltpu.async_copy(ltail_hbm.at[idx8], tail8, semt)
            da.wait()
            db.wait()
            for rr in range(8):
                rbase = jnp.full((16,), rr, jnp.int32)

                def colloop(j, _):
                    for u in range(8):
                        cidx = maskp_v[pl.ds((j * 8 + u) * 16, 16)]
                        vals = plsc.load_gather(rows8, [rbase, cidx])
                        outblk[rr, pl.ds((j * 8 + u) * 16, 16)] = vals
                    return 0

                lax.fori_loop(0, _SAFE // 128, colloop, 0)

                def colloop2(j, _):
                    cidx = maskp_v[pl.ds(j * 16, 16)]
                    wi = jnp.minimum(cidx, _WID - 1)
                    ti = jnp.maximum(cidx, _TB) - _TB
                    va = plsc.load_gather(rows8, [rbase, wi])
                    vb = plsc.load_gather(tail8, [rbase, ti])
                    vals = jnp.where(cidx < _WID, va, vb)
                    outblk[rr, pl.ds(j * 16, 16)] = vals
                    return 0

                lax.fori_loop(_SAFE // 16, NM // 16, colloop2, 0)
                cidx = maskp_v[pl.ds(NM - NM % 16, 16)]
                wi = jnp.minimum(cidx, _WID - 1)
                ti = jnp.maximum(cidx, _TB) - _TB
                va = plsc.load_gather(rows8, [rbase, wi])
                vb = plsc.load_gather(tail8, [rbase, ti])
                vals = jnp.where(cidx < _WID, va, vb)
                plsc.store_scatter(outblk, [rbase, (NM - NM % 16) + lane],
                                   vals, mask=lane < (NM % 16))
            pltpu.sync_copy(outblk, lab_hbm.at[pl.ds(g * 8, 8)])
        return 0

    lax.fori_loop(0, (_NGRP8 + _NW - 1) // _NW, group, 0)


_lab_call = pl.kernel(
    _lab_body, mesh=_sc_mesh,
    compiler_params=pltpu.CompilerParams(needs_layout_passes=False),
    out_type=jax.ShapeDtypeStruct((NM, NM), jnp.float32),
    scratch_types=[
        pltpu.VMEM((_LP,), jnp.int32),
        pltpu.VMEM((8, _WID), jnp.float32),
        pltpu.VMEM((8, 128), jnp.float32),
        pltpu.VMEM((8, NM), jnp.float32),
        pltpu.SemaphoreType.DMA,
        pltpu.SemaphoreType.DMA,
    ],
)


def _hrows_body(h1_hbm, h2_hbm, maskp_hbm, o1_hbm, o2_hbm,
                idx_v, buf, sem):
    c = lax.axis_index("c")
    s = lax.axis_index("s")
    w = c * _NS + s
    rows_per_w = _LP // _NW
    base = w * rows_per_w
    pltpu.sync_copy(maskp_hbm.at[pl.ds(base, rows_per_w)], idx_v)
    for t, (h, o) in enumerate(((h1_hbm, o1_hbm), (h2_hbm, o2_hbm))):
        for half in range(rows_per_w // _KC):
            pltpu.async_copy(
                h.at[idx_v.at[pl.ds(half * _KC, _KC)]], buf, sem).wait()
            pltpu.sync_copy(buf, o.at[pl.ds(base + half * _KC, _KC)])


_hrows_call = pl.kernel(
    _hrows_body, mesh=_sc_mesh,
    out_type=[jax.ShapeDtypeStruct((_LP, D_OUT), jnp.float32),
              jax.ShapeDtypeStruct((_LP, D_OUT), jnp.float32)],
    scratch_types=[
        pltpu.VMEM((_LP // _NW,), jnp.int32),
        pltpu.VMEM((_KC, D_OUT), jnp.float32),
        pltpu.SemaphoreType.DMA,
    ],
)


_BI = 512
_BJ = 512


def _sim_body(z1i_ref, z2i_ref, z1j_ref, z2j_ref, lab_ref, out_ref):
    j = pl.program_id(1)
    a1 = z1i_ref[...]
    a2 = z2i_ref[...]
    b1 = z1j_ref[...]
    b2 = z2j_ref[...]
    col = j * _BJ + lax.broadcasted_iota(jnp.int32, (1, _BJ), 1)
    valid = col < NM
    lab = jnp.where(valid, lab_ref[...], 0.0)

    inv_tau = 1.0 / TAU

    def emat(a, b):
        p = jnp.exp(
            lax.dot_general(a, b, (((1,), (1,)), ((), ())),
                            preferred_element_type=jnp.float32) * inv_tau)
        return jnp.where(valid, p, 0.0)

    p11 = emat(a1, b1)
    p12 = emat(a1, b2)
    p22 = emat(a2, b2)
    p21 = emat(a2, b1)
    rows = jnp.stack([
        jnp.sum(p11, axis=1), jnp.sum(p11 * lab, axis=1),
        jnp.sum(p12, axis=1), jnp.sum(p12 * lab, axis=1),
        jnp.sum(p22, axis=1), jnp.sum(p22 * lab, axis=1),
        jnp.sum(p21, axis=1), jnp.sum(p21 * lab, axis=1),
    ])

    @pl.when(j == 0)
    def _():
        out_ref[...] = jnp.zeros_like(out_ref)

    out_ref[...] += rows


def _sim_sums(z1, z2, lab):
    ni = pl.cdiv(NM, _BI)
    nj = pl.cdiv(NM, _BJ)
    return pl.pallas_call(
        _sim_body,
        grid=(ni, nj),
        in_specs=[
            pl.BlockSpec((_BI, D_OUT), lambda i, j: (i, 0)),
            pl.BlockSpec((_BI, D_OUT), lambda i, j: (i, 0)),
            pl.BlockSpec((_BJ, D_OUT), lambda i, j: (j, 0)),
            pl.BlockSpec((_BJ, D_OUT), lambda i, j: (j, 0)),
            pl.BlockSpec((_BI, _BJ), lambda i, j: (i, j)),
        ],
        out_specs=pl.BlockSpec((8, _BI), lambda i, j: (0, i)),
        out_shape=jax.ShapeDtypeStruct((8, NM), jnp.float32),
    )(z1, z2, z1, z2, lab)


def _bn(x, gamma, beta):
    mean = jnp.mean(x, axis=0)
    var = jnp.mean((x - mean) ** 2, axis=0)
    return gamma * (x - mean) / jnp.sqrt(var + EPS_BN) + beta


def _prelu(x, a):
    return jnp.where(x >= 0, x, a * x)


def _l2norm_rows(z):
    n = jnp.sqrt(jnp.sum(z * z, axis=1, keepdims=True))
    return z / jnp.maximum(n, EPS_NORM)


def kernel(feat, label, edge_index, enc_mask_token,
           W1, b1, gamma1, beta1, a1,
           W2, b2, gamma2, beta2, a2,
           Wp1, bp1, Wp2, bp2):
    src = edge_index[0]
    dst = edge_index[1]

    perm = jax.random.permutation(jax.random.key(1), N)
    mask = jnp.sort(perm[:NM]).astype(jnp.int32)
    is_masked = jnp.zeros((N,), jnp.float32).at[mask].set(1.0)

    x = jnp.where(is_masked[:, None] > 0, enc_mask_token[0][None, :], feat)

    sidx32 = src.reshape(_NW, _NCH, _KC)
    didx32 = dst.reshape(_NW, _NCH, _KC)
    hs, hd = _deg_call(sidx32, didx32)
    out_deg = jnp.sum(hs, axis=0)
    in_deg = jnp.sum(hd, axis=0)
    norm_src = jnp.clip(out_deg, 1.0, None) ** (-0.5)
    norm_dst = jnp.clip(in_deg, 1.0, None) ** (-0.5)

    sidx16 = src.reshape(_NS, _NPIECE, _NPC, _MKC)
    didx16 = dst.reshape(_NS, _NPIECE, _NPC, _MKC)
    mp_zeros = jnp.zeros((8, D_IN), jnp.float32)

    def bn_b(z, gamma, beta):
        mean = jnp.mean(z, axis=1, keepdims=True)
        var = jnp.mean((z - mean) ** 2, axis=1, keepdims=True)
        return gamma * (z - mean) / jnp.sqrt(var + EPS_BN) + beta

    ns = norm_src[None, :, None]
    nd = norm_dst[None, :, None]
    tables0 = jnp.stack([x, feat]) * ns

    def scan_body(tables, l):
        parts = _mp_call(tables, sidx16, didx16, mp_zeros)
        agg = jnp.concatenate([parts[:, 0], parts[:, 1]], axis=1) * nd

        def layer1(_):
            g = jnp.einsum('bnd,dh->bnh', agg, W1,
                           preferred_element_type=jnp.float32) + b1
            g = _prelu(bn_b(g, gamma1, beta1), a1)
            u = jnp.einsum('bnh,hd->bnd', g, W2,
                           preferred_element_type=jnp.float32) * ns
            return u, jnp.zeros((2, N, D_OUT), jnp.float32)

        def layer2(_):
            h = agg + b2
            return tables, _prelu(bn_b(h, gamma2, beta2), a2)

        return lax.cond(l == 0, layer1, layer2, None)

    _, outs = lax.scan(scan_body, tables0, jnp.arange(2))
    h1 = outs[1, 0]
    h2 = outs[1, 1]

    def proj(h):
        return jnp.maximum(h @ Wp1 + bp1, 0.0) @ Wp2 + bp2

    maskp = jnp.concatenate(
        [mask, jnp.zeros((_LP - NM,), jnp.int32)])
    g1, g2 = _hrows_call(h1, h2, maskp)
    c_h = proj(g2[:NM])
    c_m = proj(g1[:NM])
    z1 = _l2norm_rows(c_h)
    z2 = _l2norm_rows(c_m)

    ltail = lax.slice(label, (0, _TB), (N, N))
    lab = _lab_call(label, ltail, maskp)

    sums = _sim_sums(z1, z2, lab)
    s11, l11, s12, l12, s22, l22, s21, l21 = [sums[k] for k in range(8)]
    d1 = jnp.exp(jnp.sum(z1 * z1, axis=1) / TAU)
    d2 = jnp.exp(jnp.sum(z2 * z2, axis=1) / TAU)

    loss1 = -jnp.log((l12 + l11 - d1) / (s11 + s12 - d1))
    loss2 = -jnp.log((l21 + l22 - d2) / (s22 + s21 - d2))
    return jnp.mean((loss1 + loss2) * 0.5)
